# baseline (speedup 1.0000x reference)
import functools

import jax
import jax.numpy as jnp
from jax import lax
from jax.experimental import pallas as pl
from jax.experimental.pallas import tpu as pltpu
from jax.experimental.pallas import tpu_sc as plsc

N = 10000
E = 320000
D = 128
HOPS = 3
ALPHA = 0.5

NC = 2
NS = 16
NW = NC * NS
CH = 72
NCHUNK = 144
E_PAD = NW * NCHUNK * CH
N_PAD = 10112
RPS = N_PAD // NS
L = 16

_mesh = plsc.VectorSubcoreMesh(core_axis_name="c", subcore_axis_name="s")

_ibuf_types = dict(
    src_i=pltpu.VMEM((CH,), jnp.int32),
    dst_i=pltpu.VMEM((CH,), jnp.int32),
    vals=pltpu.VMEM((CH * L,), jnp.float32),
    s_idx=pltpu.SemaphoreType.DMA,
)

_rbuf_types = dict(
    rows=pltpu.VMEM((CH, D), jnp.float32),
    s_rows=pltpu.SemaphoreType.DMA,
    s_scat=pltpu.SemaphoreType.DMA,
)


@functools.partial(
    pl.kernel,
    out_type=jax.ShapeDtypeStruct((NC, N_PAD, D), jnp.float32),
    mesh=_mesh,
    scratch_types=[
        pltpu.VMEM_SHARED((N_PAD, D), jnp.float32),
        dict(_ibuf_types),
        dict(_ibuf_types),
        dict(_ibuf_types),
        dict(_ibuf_types),
        dict(_rbuf_types),
        dict(_rbuf_types),
        dict(_rbuf_types),
    ],
)
def _hop_kernel(h_hbm, src_hbm, dst_hbm, val_hbm, out_hbm, acc,
                b0, b1, b2, b3, r0, r1, r2):
    cid = lax.axis_index("c")
    sid = lax.axis_index("s")
    wid = cid * NS + sid

    zero = jnp.zeros((L,), jnp.float32)

    @pl.loop(0, CH)
    def _(e):
        for c in range(D // L):
            r0["rows"][e, pl.ds(c * L, L)] = zero

    @pl.loop(0, RPS // CH)
    def _(b):
        pltpu.sync_copy(r0["rows"], acc.at[pl.ds(sid * RPS + b * CH, CH)])

    rem = RPS - (RPS // CH) * CH
    if rem:
        pltpu.sync_copy(r0["rows"].at[pl.ds(0, rem)],
                        acc.at[pl.ds(sid * RPS + RPS - rem, rem)])

    plsc.subcore_barrier()

    def issue_idx(j, buf):
        pltpu.async_copy(src_hbm.at[wid, j], buf["src_i"], buf["s_idx"])
        pltpu.async_copy(dst_hbm.at[wid, j], buf["dst_i"], buf["s_idx"])
        pltpu.async_copy(val_hbm.at[wid, j], buf["vals"], buf["s_idx"])

    def wait_idx(j, buf):
        pltpu.make_async_copy(src_hbm.at[wid, j], buf["src_i"], buf["s_idx"]).wait()
        pltpu.make_async_copy(dst_hbm.at[wid, j], buf["dst_i"], buf["s_idx"]).wait()
        pltpu.make_async_copy(val_hbm.at[wid, j], buf["vals"], buf["s_idx"]).wait()

    def issue_gather(buf, rbuf):
        pltpu.async_copy(h_hbm.at[buf["src_i"]], rbuf["rows"], rbuf["s_rows"])

    def wait_scat(buf, rbuf):
        pltpu.make_async_copy(rbuf["rows"], acc.at[buf["dst_i"]],
                              rbuf["s_scat"]).wait()

    issue_idx(0, b0)
    issue_idx(1, b1)
    wait_idx(0, b0)
    issue_gather(b0, r0)

    def process(j, bcur, bn1, bn2, rcur, rn1):
        @pl.when(j + 1 < NCHUNK)
        def _():
            wait_idx(j + 1, bn1)

        @pl.when(j >= 2)
        def _():
            wait_scat(bn2, rn1)

        @pl.when(j + 1 < NCHUNK)
        def _():
            issue_gather(bn1, rn1)

        pltpu.make_async_copy(h_hbm.at[bcur["src_i"]], rcur["rows"],
                              rcur["s_rows"]).wait()
        rows = rcur["rows"]
        vals = bcur["vals"]

        @pl.loop(0, CH)
        def _(e):
            v = vals[pl.ds(e * L, L)]
            for c in range(D // L):
                sl = pl.ds(c * L, L)
                rows[e, sl] = rows[e, sl] * v

        pltpu.async_copy(rows, acc.at[bcur["dst_i"]], rcur["s_scat"],
                         add=True)

        @pl.when(j + 2 < NCHUNK)
        def _():
            issue_idx(j + 2, bn2)

    @pl.loop(0, NCHUNK, step=12)
    def _(j):
        process(j + 0, b0, b1, b2, r0, r1)
        process(j + 1, b1, b2, b3, r1, r2)
        process(j + 2, b2, b3, b0, r2, r0)
        process(j + 3, b3, b0, b1, r0, r1)
        process(j + 4, b0, b1, b2, r1, r2)
        process(j + 5, b1, b2, b3, r2, r0)
        process(j + 6, b2, b3, b0, r0, r1)
        process(j + 7, b3, b0, b1, r1, r2)
        process(j + 8, b0, b1, b2, r2, r0)
        process(j + 9, b1, b2, b3, r0, r1)
        process(j + 10, b2, b3, b0, r1, r2)
        process(j + 11, b3, b0, b1, r2, r0)

    wait_scat(b2, r1)
    wait_scat(b3, r2)
    plsc.subcore_barrier()
    pltpu.sync_copy(acc.at[pl.ds(sid * RPS, RPS)],
                    out_hbm.at[cid, pl.ds(sid * RPS, RPS)])


_wbuf_types = dict(
    w2d=pltpu.VMEM((CH, D), jnp.float32),
    s_scat=pltpu.SemaphoreType.DMA,
)

_dbuf_types = dict(
    dst_i=pltpu.VMEM((CH,), jnp.int32),
    vals=pltpu.VMEM((CH * L,), jnp.float32),
    s_idx=pltpu.SemaphoreType.DMA,
)


@functools.partial(
    pl.kernel,
    out_type=jax.ShapeDtypeStruct((NC, N_PAD, D), jnp.float32),
    mesh=_mesh,
    scratch_types=[
        pltpu.VMEM_SHARED((N_PAD, D), jnp.float32),
        dict(_dbuf_types),
        dict(_dbuf_types),
        dict(_dbuf_types),
        dict(_wbuf_types),
        dict(_wbuf_types),
    ],
)
def _denom_kernel(dst_hbm, val_hbm, out_hbm, acc, b0, b1, b2, w0, w1):
    cid = lax.axis_index("c")
    sid = lax.axis_index("s")
    wid = cid * NS + sid

    zero = jnp.zeros((L,), jnp.float32)

    @pl.loop(0, CH)
    def _(e):
        for c in range(D // L):
            w0["w2d"][e, pl.ds(c * L, L)] = zero

    @pl.loop(0, RPS // CH)
    def _(b):
        pltpu.sync_copy(w0["w2d"], acc.at[pl.ds(sid * RPS + b * CH, CH)])

    rem = RPS - (RPS // CH) * CH
    if rem:
        pltpu.sync_copy(w0["w2d"].at[pl.ds(0, rem)],
                        acc.at[pl.ds(sid * RPS + RPS - rem, rem)])

    plsc.subcore_barrier()

    def issue_idx(j, buf):
        pltpu.async_copy(dst_hbm.at[wid, j], buf["dst_i"], buf["s_idx"])
        pltpu.async_copy(val_hbm.at[wid, j], buf["vals"], buf["s_idx"])

    def wait_idx(j, buf):
        pltpu.make_async_copy(dst_hbm.at[wid, j], buf["dst_i"], buf["s_idx"]).wait()
        pltpu.make_async_copy(val_hbm.at[wid, j], buf["vals"], buf["s_idx"]).wait()

    def wait_scat(buf, wbuf):
        pltpu.make_async_copy(wbuf["w2d"], acc.at[buf["dst_i"]],
                              wbuf["s_scat"]).wait()

    issue_idx(0, b0)
    issue_idx(1, b1)

    def process(j, bcur, bn1, bn2, wcur, wnxt):
        wait_idx(j, bcur)
        vals = bcur["vals"]
        w2d = wcur["w2d"]

        @pl.loop(0, CH)
        def _(e):
            v = vals[pl.ds(e * L, L)]
            for c in range(D // L):
                w2d[e, pl.ds(c * L, L)] = v

        pltpu.async_copy(w2d, acc.at[bcur["dst_i"]], wcur["s_scat"],
                         add=True)

        @pl.when(j >= 1)
        def _():
            wait_scat(bn2, wnxt)

        @pl.when(j + 2 < NCHUNK)
        def _():
            issue_idx(j + 2, bn2)

    @pl.loop(0, NCHUNK, step=6)
    def _(j):
        process(j, b0, b1, b2, w0, w1)
        process(j + 1, b1, b2, b0, w1, w0)
        process(j + 2, b2, b0, b1, w0, w1)
        process(j + 3, b0, b1, b2, w1, w0)
        process(j + 4, b1, b2, b0, w0, w1)
        process(j + 5, b2, b0, b1, w1, w0)

    wait_scat(b2, w1)
    plsc.subcore_barrier()
    pltpu.sync_copy(acc.at[pl.ds(sid * RPS, RPS)],
                    out_hbm.at[cid, pl.ds(sid * RPS, RPS)])


def _blend(h, p0, p1, d0, d1):
    def body(h_ref, p0_ref, p1_ref, d0_ref, d1_ref, o_ref):
        d = jnp.maximum(d0_ref[...] + d1_ref[...], 1e-12)
        agg = (p0_ref[...] + p1_ref[...]) / d
        o_ref[...] = ALPHA * h_ref[...] + (1.0 - ALPHA) * agg

    blk = N_PAD // 8
    return pl.pallas_call(
        body,
        out_shape=jax.ShapeDtypeStruct((N_PAD, D), jnp.float32),
        grid=(8,),
        in_specs=[pl.BlockSpec((blk, D), lambda i: (i, 0))] * 5,
        out_specs=pl.BlockSpec((blk, D), lambda i: (i, 0)),
    )(h, p0, p1, d0, d1)


def kernel(H, edge_index, edge_weight):
    src = edge_index[0]
    dst = edge_index[1]
    pad = E_PAD - E
    pad_idx = jnp.arange(pad, dtype=jnp.int32)
    src_p = jnp.concatenate([src.astype(jnp.int32), pad_idx % N])
    dst_p = jnp.concatenate([dst.astype(jnp.int32), N + pad_idx % (N_PAD - N)])
    w_p = jnp.concatenate([edge_weight.astype(jnp.float32),
                           jnp.zeros((pad,), jnp.float32)])
    src3 = src_p.reshape(NW, NCHUNK, CH)
    dst3 = dst_p.reshape(NW, NCHUNK, CH)
    w16 = jnp.broadcast_to(
        w_p.reshape(NW, NCHUNK, CH, 1), (NW, NCHUNK, CH, L)
    ).reshape(NW, NCHUNK, CH * L)
    h_pad = jnp.pad(H.astype(jnp.float32), ((0, N_PAD - N), (0, 0)))

    pden = _denom_kernel(dst3, w16)

    hw = h_pad
    for _ in range(HOPS):
        p = _hop_kernel(hw, src3, dst3, w16)
        hw = _blend(hw, p[0], p[1], pden[0], pden[1])
    return hw[:N].astype(H.dtype)


# speedup vs baseline: 1.1037x; 1.1037x over previous
import functools

import jax
import jax.numpy as jnp
from jax import lax
from jax.experimental import pallas as pl
from jax.experimental.pallas import tpu as pltpu
from jax.experimental.pallas import tpu_sc as plsc

N = 10000
E = 320000
D = 128
HOPS = 3
ALPHA = 0.5

NC = 2
NS = 16
NW = NC * NS
CH = 96
NCHUNK = 108
E_PAD = NW * NCHUNK * CH
N_PAD = 10240
RPS = N_PAD // NS
L = 16

_mesh = plsc.VectorSubcoreMesh(core_axis_name="c", subcore_axis_name="s")

_ibuf_types = dict(
    src_i=pltpu.VMEM((CH,), jnp.int32),
    dst_i=pltpu.VMEM((CH,), jnp.int32),
    vals=pltpu.VMEM((CH * L,), jnp.float32),
    s_idx=pltpu.SemaphoreType.DMA,
)

_rbuf_types = dict(
    rows=pltpu.VMEM((CH, D), jnp.float32),
    s_rows=pltpu.SemaphoreType.DMA,
    s_scat=pltpu.SemaphoreType.DMA,
)


@functools.partial(
    pl.kernel,
    out_type=jax.ShapeDtypeStruct((NC, N_PAD, D), jnp.float32),
    mesh=_mesh,
    scratch_types=[
        pltpu.VMEM_SHARED((N_PAD, D), jnp.float32),
        dict(_ibuf_types),
        dict(_ibuf_types),
        dict(_ibuf_types),
        dict(_rbuf_types),
        dict(_rbuf_types),
    ],
)
def _hop_kernel(h_hbm, src_hbm, dst_hbm, val_hbm, out_hbm, acc,
                b0, b1, b2, r0, r1):
    cid = lax.axis_index("c")
    sid = lax.axis_index("s")
    wid = cid * NS + sid

    zero = jnp.zeros((L,), jnp.float32)

    @pl.loop(0, CH)
    def _(e):
        for c in range(D // L):
            r0["rows"][e, pl.ds(c * L, L)] = zero

    @pl.loop(0, RPS // CH)
    def _(b):
        pltpu.sync_copy(r0["rows"], acc.at[pl.ds(sid * RPS + b * CH, CH)])

    rem = RPS - (RPS // CH) * CH
    if rem:
        pltpu.sync_copy(r0["rows"].at[pl.ds(0, rem)],
                        acc.at[pl.ds(sid * RPS + RPS - rem, rem)])

    plsc.subcore_barrier()

    def issue_idx(j, buf):
        pltpu.async_copy(src_hbm.at[wid, j], buf["src_i"], buf["s_idx"])
        pltpu.async_copy(dst_hbm.at[wid, j], buf["dst_i"], buf["s_idx"])
        pltpu.async_copy(val_hbm.at[wid, j], buf["vals"], buf["s_idx"])

    def wait_idx(j, buf):
        pltpu.make_async_copy(src_hbm.at[wid, j], buf["src_i"], buf["s_idx"]).wait()
        pltpu.make_async_copy(dst_hbm.at[wid, j], buf["dst_i"], buf["s_idx"]).wait()
        pltpu.make_async_copy(val_hbm.at[wid, j], buf["vals"], buf["s_idx"]).wait()

    def issue_gather(buf, rbuf):
        pltpu.async_copy(h_hbm.at[buf["src_i"]], rbuf["rows"], rbuf["s_rows"])

    def wait_scat(buf, rbuf):
        pltpu.make_async_copy(rbuf["rows"], acc.at[buf["dst_i"]],
                              rbuf["s_scat"]).wait()

    issue_idx(0, b0)
    issue_idx(1, b1)
    wait_idx(0, b0)
    issue_gather(b0, r0)

    def process(j, bcur, bn1, bn2, rcur, rn1):
        @pl.when(j + 1 < NCHUNK)
        def _():
            wait_idx(j + 1, bn1)

        @pl.when(j >= 1)
        def _():
            wait_scat(bn2, rn1)

        @pl.when(j + 1 < NCHUNK)
        def _():
            issue_gather(bn1, rn1)

        pltpu.make_async_copy(h_hbm.at[bcur["src_i"]], rcur["rows"],
                              rcur["s_rows"]).wait()
        rows = rcur["rows"]
        vals = bcur["vals"]

        @pl.loop(0, CH)
        def _(e):
            v = vals[pl.ds(e * L, L)]
            for c in range(D // L):
                sl = pl.ds(c * L, L)
                rows[e, sl] = rows[e, sl] * v

        pltpu.async_copy(rows, acc.at[bcur["dst_i"]], rcur["s_scat"],
                         add=True)

        @pl.when(j + 2 < NCHUNK)
        def _():
            issue_idx(j + 2, bn2)

    @pl.loop(0, NCHUNK, step=6)
    def _(j):
        process(j, b0, b1, b2, r0, r1)
        process(j + 1, b1, b2, b0, r1, r0)
        process(j + 2, b2, b0, b1, r0, r1)
        process(j + 3, b0, b1, b2, r1, r0)
        process(j + 4, b1, b2, b0, r0, r1)
        process(j + 5, b2, b0, b1, r1, r0)

    wait_scat(b2, r1)
    plsc.subcore_barrier()
    pltpu.sync_copy(acc.at[pl.ds(sid * RPS, RPS)],
                    out_hbm.at[cid, pl.ds(sid * RPS, RPS)])


_wbuf_types = dict(
    w2d=pltpu.VMEM((CH, D), jnp.float32),
    s_scat=pltpu.SemaphoreType.DMA,
)

_dbuf_types = dict(
    dst_i=pltpu.VMEM((CH,), jnp.int32),
    vals=pltpu.VMEM((CH * L,), jnp.float32),
    s_idx=pltpu.SemaphoreType.DMA,
)


@functools.partial(
    pl.kernel,
    out_type=jax.ShapeDtypeStruct((NC, N_PAD, D), jnp.float32),
    mesh=_mesh,
    scratch_types=[
        pltpu.VMEM_SHARED((N_PAD, D), jnp.float32),
        dict(_dbuf_types),
        dict(_dbuf_types),
        dict(_dbuf_types),
        dict(_wbuf_types),
        dict(_wbuf_types),
    ],
)
def _denom_kernel(dst_hbm, val_hbm, out_hbm, acc, b0, b1, b2, w0, w1):
    cid = lax.axis_index("c")
    sid = lax.axis_index("s")
    wid = cid * NS + sid

    zero = jnp.zeros((L,), jnp.float32)

    @pl.loop(0, CH)
    def _(e):
        for c in range(D // L):
            w0["w2d"][e, pl.ds(c * L, L)] = zero

    @pl.loop(0, RPS // CH)
    def _(b):
        pltpu.sync_copy(w0["w2d"], acc.at[pl.ds(sid * RPS + b * CH, CH)])

    rem = RPS - (RPS // CH) * CH
    if rem:
        pltpu.sync_copy(w0["w2d"].at[pl.ds(0, rem)],
                        acc.at[pl.ds(sid * RPS + RPS - rem, rem)])

    plsc.subcore_barrier()

    def issue_idx(j, buf):
        pltpu.async_copy(dst_hbm.at[wid, j], buf["dst_i"], buf["s_idx"])
        pltpu.async_copy(val_hbm.at[wid, j], buf["vals"], buf["s_idx"])

    def wait_idx(j, buf):
        pltpu.make_async_copy(dst_hbm.at[wid, j], buf["dst_i"], buf["s_idx"]).wait()
        pltpu.make_async_copy(val_hbm.at[wid, j], buf["vals"], buf["s_idx"]).wait()

    def wait_scat(buf, wbuf):
        pltpu.make_async_copy(wbuf["w2d"], acc.at[buf["dst_i"]],
                              wbuf["s_scat"]).wait()

    issue_idx(0, b0)
    issue_idx(1, b1)

    def process(j, bcur, bn1, bn2, wcur, wnxt):
        wait_idx(j, bcur)
        vals = bcur["vals"]
        w2d = wcur["w2d"]

        @pl.loop(0, CH)
        def _(e):
            v = vals[pl.ds(e * L, L)]
            for c in range(D // L):
                w2d[e, pl.ds(c * L, L)] = v

        pltpu.async_copy(w2d, acc.at[bcur["dst_i"]], wcur["s_scat"],
                         add=True)

        @pl.when(j >= 1)
        def _():
            wait_scat(bn2, wnxt)

        @pl.when(j + 2 < NCHUNK)
        def _():
            issue_idx(j + 2, bn2)

    @pl.loop(0, NCHUNK, step=6)
    def _(j):
        process(j, b0, b1, b2, w0, w1)
        process(j + 1, b1, b2, b0, w1, w0)
        process(j + 2, b2, b0, b1, w0, w1)
        process(j + 3, b0, b1, b2, w1, w0)
        process(j + 4, b1, b2, b0, w0, w1)
        process(j + 5, b2, b0, b1, w1, w0)

    wait_scat(b2, w1)
    plsc.subcore_barrier()
    pltpu.sync_copy(acc.at[pl.ds(sid * RPS, RPS)],
                    out_hbm.at[cid, pl.ds(sid * RPS, RPS)])


def _blend(h, p0, p1, d0, d1):
    def body(h_ref, p0_ref, p1_ref, d0_ref, d1_ref, o_ref):
        d = jnp.maximum(d0_ref[...] + d1_ref[...], 1e-12)
        agg = (p0_ref[...] + p1_ref[...]) / d
        o_ref[...] = ALPHA * h_ref[...] + (1.0 - ALPHA) * agg

    blk = N_PAD // 8
    return pl.pallas_call(
        body,
        out_shape=jax.ShapeDtypeStruct((N_PAD, D), jnp.float32),
        grid=(8,),
        in_specs=[pl.BlockSpec((blk, D), lambda i: (i, 0))] * 5,
        out_specs=pl.BlockSpec((blk, D), lambda i: (i, 0)),
    )(h, p0, p1, d0, d1)


def kernel(H, edge_index, edge_weight):
    src = edge_index[0]
    dst = edge_index[1]
    pad = E_PAD - E
    pad_idx = jnp.arange(pad, dtype=jnp.int32)
    src_p = jnp.concatenate([src.astype(jnp.int32), pad_idx % N])
    dst_p = jnp.concatenate([dst.astype(jnp.int32), N + pad_idx % (N_PAD - N)])
    w_p = jnp.concatenate([edge_weight.astype(jnp.float32),
                           jnp.zeros((pad,), jnp.float32)])
    src3 = src_p.reshape(NW, NCHUNK, CH)
    dst3 = dst_p.reshape(NW, NCHUNK, CH)
    w16 = jnp.broadcast_to(
        w_p.reshape(NW, NCHUNK, CH, 1), (NW, NCHUNK, CH, L)
    ).reshape(NW, NCHUNK, CH * L)
    h_pad = jnp.pad(H.astype(jnp.float32), ((0, N_PAD - N), (0, 0)))

    pden = _denom_kernel(dst3, w16)

    hw = h_pad
    for _ in range(HOPS):
        p = _hop_kernel(hw, src3, dst3, w16)
        hw = _blend(hw, p[0], p[1], pden[0], pden[1])
    return hw[:N].astype(H.dtype)


# speedup vs baseline: 1.1682x; 1.0584x over previous
import functools

import jax
import jax.numpy as jnp
from jax import lax
from jax.experimental import pallas as pl
from jax.experimental.pallas import tpu as pltpu
from jax.experimental.pallas import tpu_sc as plsc

N = 10000
E = 320000
D = 128
HOPS = 3
ALPHA = 0.5

NC = 2
NS = 16
NW = NC * NS
CH = 96
NCHUNK = 108
E_PAD = NW * NCHUNK * CH
N_PAD = 10240
RPS = N_PAD // NS
L = 16

_mesh = plsc.VectorSubcoreMesh(core_axis_name="c", subcore_axis_name="s")

_ibuf_types = dict(
    src_i=pltpu.VMEM((CH,), jnp.int32),
    dst_i=pltpu.VMEM((CH,), jnp.int32),
    vals=pltpu.VMEM((CH,), jnp.float32),
    s_idx=pltpu.SemaphoreType.DMA,
)

_rbuf_types = dict(
    rows=pltpu.VMEM((CH, D), jnp.float32),
    s_rows=pltpu.SemaphoreType.DMA,
    s_scat=pltpu.SemaphoreType.DMA,
)


@functools.partial(
    pl.kernel,
    out_type=jax.ShapeDtypeStruct((NC, N_PAD, D), jnp.float32),
    mesh=_mesh,
    compiler_params=pltpu.CompilerParams(needs_layout_passes=False),
    scratch_types=[
        pltpu.VMEM_SHARED((N_PAD, D), jnp.float32),
        dict(_ibuf_types),
        dict(_ibuf_types),
        dict(_ibuf_types),
        dict(_rbuf_types),
        dict(_rbuf_types),
    ],
)
def _hop_kernel(h_hbm, src_hbm, dst_hbm, val_hbm, out_hbm, acc,
                b0, b1, b2, r0, r1):
    cid = lax.axis_index("c")
    sid = lax.axis_index("s")
    wid = cid * NS + sid

    zero = jnp.zeros((L,), jnp.float32)

    @pl.loop(0, CH)
    def _(e):
        for c in range(D // L):
            r0["rows"][e, pl.ds(c * L, L)] = zero

    @pl.loop(0, RPS // CH)
    def _(b):
        pltpu.sync_copy(r0["rows"], acc.at[pl.ds(sid * RPS + b * CH, CH)])

    rem = RPS - (RPS // CH) * CH
    if rem:
        pltpu.sync_copy(r0["rows"].at[pl.ds(0, rem)],
                        acc.at[pl.ds(sid * RPS + RPS - rem, rem)])

    plsc.subcore_barrier()

    def issue_idx(j, buf):
        pltpu.async_copy(src_hbm.at[wid, j], buf["src_i"], buf["s_idx"])
        pltpu.async_copy(dst_hbm.at[wid, j], buf["dst_i"], buf["s_idx"])
        pltpu.async_copy(val_hbm.at[wid, j], buf["vals"], buf["s_idx"])

    def wait_idx(j, buf):
        pltpu.make_async_copy(src_hbm.at[wid, j], buf["src_i"], buf["s_idx"]).wait()
        pltpu.make_async_copy(dst_hbm.at[wid, j], buf["dst_i"], buf["s_idx"]).wait()
        pltpu.make_async_copy(val_hbm.at[wid, j], buf["vals"], buf["s_idx"]).wait()

    def issue_gather(buf, rbuf):
        pltpu.async_copy(h_hbm.at[buf["src_i"]], rbuf["rows"], rbuf["s_rows"])

    def wait_scat(buf, rbuf):
        pltpu.make_async_copy(rbuf["rows"], acc.at[buf["dst_i"]],
                              rbuf["s_scat"]).wait()

    issue_idx(0, b0)
    issue_idx(1, b1)
    wait_idx(0, b0)
    issue_gather(b0, r0)

    def process(j, bcur, bn1, bn2, rcur, rn1):
        @pl.when(j + 1 < NCHUNK)
        def _():
            wait_idx(j + 1, bn1)

        @pl.when(j >= 1)
        def _():
            wait_scat(bn2, rn1)

        @pl.when(j + 1 < NCHUNK)
        def _():
            issue_gather(bn1, rn1)

        pltpu.make_async_copy(h_hbm.at[bcur["src_i"]], rcur["rows"],
                              rcur["s_rows"]).wait()
        rows = rcur["rows"]
        vals = bcur["vals"]

        @pl.loop(0, CH)
        def _(e):
            v = plsc.load_gather(vals, [jnp.full((L,), e, jnp.int32)])
            for c in range(D // L):
                sl = pl.ds(c * L, L)
                rows[e, sl] = rows[e, sl] * v

        pltpu.async_copy(rows, acc.at[bcur["dst_i"]], rcur["s_scat"],
                         add=True)

        @pl.when(j + 2 < NCHUNK)
        def _():
            issue_idx(j + 2, bn2)

    @pl.loop(0, NCHUNK, step=6)
    def _(j):
        process(j, b0, b1, b2, r0, r1)
        process(j + 1, b1, b2, b0, r1, r0)
        process(j + 2, b2, b0, b1, r0, r1)
        process(j + 3, b0, b1, b2, r1, r0)
        process(j + 4, b1, b2, b0, r0, r1)
        process(j + 5, b2, b0, b1, r1, r0)

    wait_scat(b2, r1)
    plsc.subcore_barrier()
    pltpu.sync_copy(acc.at[pl.ds(sid * RPS, RPS)],
                    out_hbm.at[cid, pl.ds(sid * RPS, RPS)])


_wbuf_types = dict(
    w2d=pltpu.VMEM((CH, D), jnp.float32),
    s_scat=pltpu.SemaphoreType.DMA,
)

_dbuf_types = dict(
    dst_i=pltpu.VMEM((CH,), jnp.int32),
    vals=pltpu.VMEM((CH,), jnp.float32),
    s_idx=pltpu.SemaphoreType.DMA,
)


@functools.partial(
    pl.kernel,
    out_type=jax.ShapeDtypeStruct((NC, N_PAD, D), jnp.float32),
    mesh=_mesh,
    compiler_params=pltpu.CompilerParams(needs_layout_passes=False),
    scratch_types=[
        pltpu.VMEM_SHARED((N_PAD, D), jnp.float32),
        dict(_dbuf_types),
        dict(_dbuf_types),
        dict(_dbuf_types),
        dict(_wbuf_types),
        dict(_wbuf_types),
    ],
)
def _denom_kernel(dst_hbm, val_hbm, out_hbm, acc, b0, b1, b2, w0, w1):
    cid = lax.axis_index("c")
    sid = lax.axis_index("s")
    wid = cid * NS + sid

    zero = jnp.zeros((L,), jnp.float32)

    @pl.loop(0, CH)
    def _(e):
        for c in range(D // L):
            w0["w2d"][e, pl.ds(c * L, L)] = zero

    @pl.loop(0, RPS // CH)
    def _(b):
        pltpu.sync_copy(w0["w2d"], acc.at[pl.ds(sid * RPS + b * CH, CH)])

    rem = RPS - (RPS // CH) * CH
    if rem:
        pltpu.sync_copy(w0["w2d"].at[pl.ds(0, rem)],
                        acc.at[pl.ds(sid * RPS + RPS - rem, rem)])

    plsc.subcore_barrier()

    def issue_idx(j, buf):
        pltpu.async_copy(dst_hbm.at[wid, j], buf["dst_i"], buf["s_idx"])
        pltpu.async_copy(val_hbm.at[wid, j], buf["vals"], buf["s_idx"])

    def wait_idx(j, buf):
        pltpu.make_async_copy(dst_hbm.at[wid, j], buf["dst_i"], buf["s_idx"]).wait()
        pltpu.make_async_copy(val_hbm.at[wid, j], buf["vals"], buf["s_idx"]).wait()

    def wait_scat(buf, wbuf):
        pltpu.make_async_copy(wbuf["w2d"], acc.at[buf["dst_i"]],
                              wbuf["s_scat"]).wait()

    issue_idx(0, b0)
    issue_idx(1, b1)

    def process(j, bcur, bn1, bn2, wcur, wnxt):
        wait_idx(j, bcur)
        vals = bcur["vals"]
        w2d = wcur["w2d"]

        @pl.loop(0, CH)
        def _(e):
            v = plsc.load_gather(vals, [jnp.full((L,), e, jnp.int32)])
            for c in range(D // L):
                w2d[e, pl.ds(c * L, L)] = v

        pltpu.async_copy(w2d, acc.at[bcur["dst_i"]], wcur["s_scat"],
                         add=True)

        @pl.when(j >= 1)
        def _():
            wait_scat(bn2, wnxt)

        @pl.when(j + 2 < NCHUNK)
        def _():
            issue_idx(j + 2, bn2)

    @pl.loop(0, NCHUNK, step=6)
    def _(j):
        process(j, b0, b1, b2, w0, w1)
        process(j + 1, b1, b2, b0, w1, w0)
        process(j + 2, b2, b0, b1, w0, w1)
        process(j + 3, b0, b1, b2, w1, w0)
        process(j + 4, b1, b2, b0, w0, w1)
        process(j + 5, b2, b0, b1, w1, w0)

    wait_scat(b2, w1)
    plsc.subcore_barrier()
    pltpu.sync_copy(acc.at[pl.ds(sid * RPS, RPS)],
                    out_hbm.at[cid, pl.ds(sid * RPS, RPS)])


def _blend(h, p0, p1, d0, d1):
    def body(h_ref, p0_ref, p1_ref, d0_ref, d1_ref, o_ref):
        d = jnp.maximum(d0_ref[...] + d1_ref[...], 1e-12)
        agg = (p0_ref[...] + p1_ref[...]) / d
        o_ref[...] = ALPHA * h_ref[...] + (1.0 - ALPHA) * agg

    blk = N_PAD // 8
    return pl.pallas_call(
        body,
        out_shape=jax.ShapeDtypeStruct((N_PAD, D), jnp.float32),
        grid=(8,),
        in_specs=[pl.BlockSpec((blk, D), lambda i: (i, 0))] * 5,
        out_specs=pl.BlockSpec((blk, D), lambda i: (i, 0)),
    )(h, p0, p1, d0, d1)


def kernel(H, edge_index, edge_weight):
    src = edge_index[0]
    dst = edge_index[1]
    pad = E_PAD - E
    pad_idx = jnp.arange(pad, dtype=jnp.int32)
    src_p = jnp.concatenate([src.astype(jnp.int32), pad_idx % N])
    dst_p = jnp.concatenate([dst.astype(jnp.int32), N + pad_idx % (N_PAD - N)])
    w_p = jnp.concatenate([edge_weight.astype(jnp.float32),
                           jnp.zeros((pad,), jnp.float32)])
    src3 = src_p.reshape(NW, NCHUNK, CH)
    dst3 = dst_p.reshape(NW, NCHUNK, CH)
    w3 = w_p.reshape(NW, NCHUNK, CH)
    h_pad = jnp.pad(H.astype(jnp.float32), ((0, N_PAD - N), (0, 0)))

    pden = _denom_kernel(dst3, w3)

    hw = h_pad
    for _ in range(HOPS):
        p = _hop_kernel(hw, src3, dst3, w3)
        hw = _blend(hw, p[0], p[1], pden[0], pden[1])
    return hw[:N].astype(H.dtype)
